# baseline (speedup 1.0000x reference)
import functools

import jax
import jax.numpy as jnp
from jax import lax
from jax.experimental import pallas as pl
from jax.experimental.pallas import tpu as pltpu
from jax.experimental.pallas import tpu_sc as plsc

_B = 128
_D = 64
_B_PER_W = 8
_NW_ACTIVE = _B // _B_PER_W


def _make_gather():
    mesh = plsc.VectorSubcoreMesh(core_axis_name="c", subcore_axis_name="s")
    info = plsc.get_sparse_core_info()
    num_cores = info.num_cores

    @functools.partial(
        pl.kernel,
        mesh=mesh,
        out_type=jax.ShapeDtypeStruct((_B, _D), jnp.float32),
        scratch_types=[
            pltpu.VMEM((_B_PER_W,), jnp.int32),
            pltpu.VMEM((_B_PER_W, _D), jnp.float32),
            pltpu.SemaphoreType.DMA,
        ],
        compiler_params=pltpu.CompilerParams(use_tc_tiling_on_sc=False),
    )
    def gather_kernel(table_hbm, idx_hbm, out_hbm, idx_v, rows_v, sem):
        wid = lax.axis_index("s") * num_cores + lax.axis_index("c")

        @pl.when(wid < _NW_ACTIVE)
        def _():
            base = wid * _B_PER_W
            pltpu.sync_copy(idx_hbm.at[pl.ds(base, _B_PER_W)], idx_v)
            pltpu.async_copy(table_hbm.at[idx_v], rows_v, sem).wait()
            pltpu.sync_copy(rows_v, out_hbm.at[pl.ds(base, _B_PER_W)])

    return gather_kernel


_gather = _make_gather()


def kernel(x, index):
    return _gather(x, index.astype(jnp.int32))


# speedup vs baseline: 1.7294x; 1.7294x over previous
import functools

import jax
import jax.numpy as jnp
from jax import lax
from jax.experimental import pallas as pl
from jax.experimental.pallas import tpu as pltpu
from jax.experimental.pallas import tpu_sc as plsc

_B = 128
_D = 64
_B_PER_W = 8
_NW_ACTIVE = _B // _B_PER_W


def _make_gather():
    mesh = plsc.VectorSubcoreMesh(core_axis_name="c", subcore_axis_name="s")
    info = plsc.get_sparse_core_info()
    num_cores = info.num_cores

    @functools.partial(
        pl.kernel,
        mesh=mesh,
        out_type=jax.ShapeDtypeStruct((_B, _D), jnp.float32),
        scratch_types=[
            pltpu.VMEM((_B + 16,), jnp.int32),
            pltpu.VMEM((_B_PER_W, _D), jnp.float32),
            pltpu.SemaphoreType.DMA,
        ],
        compiler_params=pltpu.CompilerParams(needs_layout_passes=False),
    )
    def gather_kernel(table_hbm, idx_hbm, out_hbm, idx_all, rows_v, sem):
        wid = lax.axis_index("s") * num_cores + lax.axis_index("c")

        @pl.when(wid < _NW_ACTIVE)
        def _():
            pltpu.sync_copy(idx_hbm, idx_all.at[pl.ds(0, _B)])
            v = idx_all[pl.ds(wid * _B_PER_W, 16)]
            copies = [
                pltpu.async_copy(table_hbm.at[pl.ds(v[j], 1)],
                                 rows_v.at[pl.ds(j, 1)], sem)
                for j in range(_B_PER_W)
            ]
            for c in copies:
                c.wait()
            pltpu.sync_copy(rows_v, out_hbm.at[pl.ds(wid * _B_PER_W, _B_PER_W)])

    return gather_kernel


_gather = _make_gather()


def kernel(x, index):
    return _gather(x, index.astype(jnp.int32))


# speedup vs baseline: 1.7387x; 1.0054x over previous
import functools

import jax
import jax.numpy as jnp
from jax import lax
from jax.experimental import pallas as pl
from jax.experimental.pallas import tpu as pltpu
from jax.experimental.pallas import tpu_sc as plsc

_B = 128
_D = 64
_B_PER_W = 8
_NW_ACTIVE = _B // _B_PER_W


def _make_gather():
    mesh = plsc.VectorSubcoreMesh(core_axis_name="c", subcore_axis_name="s")
    info = plsc.get_sparse_core_info()
    num_cores = info.num_cores

    @functools.partial(
        pl.kernel,
        mesh=mesh,
        out_type=jax.ShapeDtypeStruct((_B, _D), jnp.float32),
        scratch_types=[
            pltpu.VMEM((_B + 16,), jnp.int32),
            pltpu.VMEM((_B_PER_W, _D), jnp.float32),
            pltpu.SemaphoreType.DMA,
        ],
    )
    def gather_kernel(table_hbm, idx_hbm, out_hbm, idx_all, rows_v, sem):
        wid = lax.axis_index("s") * num_cores + lax.axis_index("c")

        @pl.when(wid < _NW_ACTIVE)
        def _():
            pltpu.sync_copy(idx_hbm, idx_all.at[pl.ds(0, _B)])
            v = idx_all[pl.ds(wid * _B_PER_W, 16)]
            copies = [
                pltpu.async_copy(table_hbm.at[pl.ds(v[j], 1)],
                                 rows_v.at[pl.ds(j, 1)], sem)
                for j in range(_B_PER_W)
            ]
            for c in copies:
                c.wait()
            pltpu.sync_copy(rows_v, out_hbm.at[pl.ds(wid * _B_PER_W, _B_PER_W)])

    return gather_kernel


_gather = _make_gather()


def kernel(x, index):
    return _gather(x, index.astype(jnp.int32))


# speedup vs baseline: 24.4567x; 14.0658x over previous
import functools

import jax
import jax.numpy as jnp
from jax import lax
from jax.experimental import pallas as pl
from jax.experimental.pallas import tpu as pltpu
from jax.experimental.pallas import tpu_sc as plsc

_B = 128
_D = 64
_LANES = 128
_B_PER_W = 8
_NW_ACTIVE = _B // _B_PER_W


def _make_gather():
    mesh = plsc.VectorSubcoreMesh(core_axis_name="c", subcore_axis_name="s")
    info = plsc.get_sparse_core_info()
    num_cores = info.num_cores

    @functools.partial(
        pl.kernel,
        mesh=mesh,
        out_type=jax.ShapeDtypeStruct((_B, _D), jnp.float32),
        scratch_types=[
            pltpu.VMEM((_B + 16,), jnp.int32),
            pltpu.VMEM((_B_PER_W, _D, _LANES), jnp.float32),
            pltpu.VMEM((_B_PER_W, _D), jnp.float32),
            pltpu.SemaphoreType.DMA,
        ],
        compiler_params=pltpu.CompilerParams(
            needs_layout_passes=False, use_tc_tiling_on_sc=True),
    )
    def gather_kernel(xt_hbm, idx_hbm, out_hbm, idx_all, tbuf, rows_v, sem):
        wid = lax.axis_index("s") * num_cores + lax.axis_index("c")

        @pl.when(wid < _NW_ACTIVE)
        def _():
            pltpu.sync_copy(idx_hbm, idx_all.at[pl.ds(0, _B)])
            v = idx_all[pl.ds(wid * _B_PER_W, 16)]
            copies = []
            for j in range(_B_PER_W):
                base = pl.multiple_of(
                    lax.shift_left(lax.shift_right_logical(v[j], 7), 7),
                    _LANES)
                copies.append(
                    pltpu.async_copy(xt_hbm.at[:, pl.ds(base, _LANES)],
                                     tbuf.at[j], sem))
            lane = lax.iota(jnp.int32, 16)
            for j in range(_B_PER_W):
                copies[j].wait()
                lane_b = jnp.broadcast_to(
                    lax.bitwise_and(v[j], _LANES - 1), (16,))
                a_j = jnp.full((16,), j, jnp.int32)
                for q in range(_D // 16):
                    rows_v[j, pl.ds(16 * q, 16)] = plsc.load_gather(
                        tbuf, [a_j, lane + (16 * q), lane_b])
            pltpu.sync_copy(rows_v,
                            out_hbm.at[pl.ds(wid * _B_PER_W, _B_PER_W)])

    return gather_kernel


_gather = _make_gather()


def kernel(x, index):
    return _gather(x.T, index.astype(jnp.int32))


# speedup vs baseline: 24.8511x; 1.0161x over previous
import functools

import jax
import jax.numpy as jnp
from jax import lax
from jax.experimental import pallas as pl
from jax.experimental.pallas import tpu as pltpu
from jax.experimental.pallas import tpu_sc as plsc

_B = 128
_D = 64
_LANES = 128
_B_PER_W = 4
_NW = _B // _B_PER_W


def _make_gather():
    mesh = plsc.VectorSubcoreMesh(core_axis_name="c", subcore_axis_name="s")
    info = plsc.get_sparse_core_info()
    num_cores = info.num_cores

    @functools.partial(
        pl.kernel,
        mesh=mesh,
        out_type=jax.ShapeDtypeStruct((_NW, _B_PER_W, _D), jnp.float32),
        scratch_types=[
            pltpu.VMEM((_B + 16,), jnp.int32),
            pltpu.VMEM((_B_PER_W, _D, _LANES), jnp.float32),
            pltpu.VMEM((_B_PER_W, _D), jnp.float32),
            pltpu.SemaphoreType.DMA,
        ],
        compiler_params=pltpu.CompilerParams(
            needs_layout_passes=False, use_tc_tiling_on_sc=True),
    )
    def gather_kernel(xt_hbm, idx_hbm, out_hbm, idx_all, tbuf, rows_v, sem):
        wid = lax.axis_index("s") * num_cores + lax.axis_index("c")
        pltpu.sync_copy(idx_hbm, idx_all.at[pl.ds(0, _B)])
        v16 = idx_all[pl.ds((wid >> 1) * 8, 16)]
        parity = lax.bitwise_and(wid, 1)
        cs = [lax.select(parity == 0, v16[j], v16[j + _B_PER_W])
              for j in range(_B_PER_W)]
        copies = []
        for j in range(_B_PER_W):
            base = pl.multiple_of(
                lax.shift_left(lax.shift_right_logical(cs[j], 7), 7), _LANES)
            copies.append(
                pltpu.async_copy(xt_hbm.at[:, pl.ds(base, _LANES)],
                                 tbuf.at[j], sem))
        lane = lax.iota(jnp.int32, 16)
        for j in range(_B_PER_W):
            copies[j].wait()
            lane_b = jnp.broadcast_to(
                lax.bitwise_and(cs[j], _LANES - 1), (16,))
            a_j = jnp.full((16,), j, jnp.int32)
            for q in range(_D // 16):
                rows_v[j, pl.ds(16 * q, 16)] = plsc.load_gather(
                    tbuf, [a_j, lane + (16 * q), lane_b])
        pltpu.sync_copy(rows_v, out_hbm.at[wid])

    return gather_kernel


_gather = _make_gather()


def kernel(x, index):
    out3 = _gather(x.T, index.astype(jnp.int32))
    return out3.reshape(_B, _D)
